# Initial kernel scaffold; baseline (speedup 1.0000x reference)
import jax
import jax.numpy as jnp
from jax.experimental import pallas as pl


def kernel(x, edge_index, edge_type, edge_pairs, W_rel0, W_root0, b0, W_rel1, W_root1, b1, Wl1, bl1, Wl2, bl2, Wl3, bl3):
    raise NotImplementedError("write your pallas kernel here")


# speedup vs baseline: 4.8947x; 4.8947x over previous
import jax
import jax.numpy as jnp
from jax import lax
from jax.experimental import pallas as pl
from jax.experimental.pallas import tpu as pltpu
from jax.experimental.pallas import tpu_sc as plsc

_N = 10000
_E = 320000
_D = 128
_H = 128
_P = 10000
_PP = 10240
_CH = 128
_NC = 2
_NS = 16
_NW = _NC * _NS
_RCH = 16
_NRC = _N // _RCH
_BLK = 1000
_PBLK = 1024

_f32 = jnp.float32


def _pre_body(x_ref, wa_ref, wb_ref, b_ref, m_ref, r_ref):
    xb = x_ref[...]
    m_ref[...] = jnp.dot(xb, wa_ref[...], preferred_element_type=_f32)
    r_ref[...] = jnp.dot(xb, wb_ref[...], preferred_element_type=_f32) + b_ref[...]


def _pre(x, wrel, wroot, b):
    return pl.pallas_call(
        _pre_body,
        grid=(_N // _BLK,),
        in_specs=[
            pl.BlockSpec((_BLK, _D), lambda i: (i, 0)),
            pl.BlockSpec((_D, _H), lambda i: (0, 0)),
            pl.BlockSpec((_D, _H), lambda i: (0, 0)),
            pl.BlockSpec((1, _H), lambda i: (0, 0)),
        ],
        out_specs=[pl.BlockSpec((_BLK, _H), lambda i: (i, 0))] * 2,
        out_shape=[jax.ShapeDtypeStruct((_N, _H), _f32)] * 2,
    )(x, wrel, wroot, b)


def _norm_h(r_ref, agg_ref, deg_ref):
    agg = agg_ref[0] + agg_ref[1]
    deg = deg_ref[0, :, 0:1] + deg_ref[1, :, 0:1]
    inv = 1.0 / jnp.maximum(deg, 1.0)
    return jnp.maximum(r_ref[...] + agg * inv, 0.0)


def _mid_body(r_ref, agg_ref, deg_ref, wa_ref, wb_ref, b_ref, m_ref, rn_ref):
    h = _norm_h(r_ref, agg_ref, deg_ref)
    m_ref[...] = jnp.dot(h, wa_ref[...], preferred_element_type=_f32)
    rn_ref[...] = jnp.dot(h, wb_ref[...], preferred_element_type=_f32) + b_ref[...]


def _mid(r0, agg, deg, wrel, wroot, b):
    return pl.pallas_call(
        _mid_body,
        grid=(_N // _BLK,),
        in_specs=[
            pl.BlockSpec((_BLK, _H), lambda i: (i, 0)),
            pl.BlockSpec((_NC, _BLK, _H), lambda i: (0, i, 0)),
            pl.BlockSpec((_NC, _BLK, _H), lambda i: (0, i, 0)),
            pl.BlockSpec((_H, _H), lambda i: (0, 0)),
            pl.BlockSpec((_H, _H), lambda i: (0, 0)),
            pl.BlockSpec((1, _H), lambda i: (0, 0)),
        ],
        out_specs=[pl.BlockSpec((_BLK, _H), lambda i: (i, 0))] * 2,
        out_shape=[jax.ShapeDtypeStruct((_N, _H), _f32)] * 2,
    )(r0, agg, deg, wrel, wroot, b)


def _fin_body(r_ref, agg_ref, deg_ref, h_ref):
    h_ref[...] = _norm_h(r_ref, agg_ref, deg_ref)


def _fin(r1, agg, deg):
    return pl.pallas_call(
        _fin_body,
        grid=(_N // _BLK,),
        in_specs=[
            pl.BlockSpec((_BLK, _H), lambda i: (i, 0)),
            pl.BlockSpec((_NC, _BLK, _H), lambda i: (0, i, 0)),
            pl.BlockSpec((_NC, _BLK, _H), lambda i: (0, i, 0)),
        ],
        out_specs=pl.BlockSpec((_BLK, _H), lambda i: (i, 0)),
        out_shape=jax.ShapeDtypeStruct((_N, _H), _f32),
    )(r1, agg, deg)


def _mlp_body(u_ref, v_ref, w1u_ref, w1v_ref, b1_ref, w2_ref, b2_ref,
              w3_ref, b3_ref, o_ref):
    z = jnp.maximum(
        jnp.dot(u_ref[...], w1u_ref[...], preferred_element_type=_f32)
        + jnp.dot(v_ref[...], w1v_ref[...], preferred_element_type=_f32)
        + b1_ref[...], 0.0)
    z = jnp.maximum(jnp.dot(z, w2_ref[...], preferred_element_type=_f32)
                    + b2_ref[...], 0.0)
    t = jnp.sum(z * w3_ref[...], axis=1, keepdims=True) + b3_ref[...]
    o_ref[...] = 1.0 / (1.0 + jnp.exp(-t))


def _mlp(u, v, w1u, w1v, b1, w2, b2, w3r, b3):
    return pl.pallas_call(
        _mlp_body,
        grid=(_PP // _PBLK,),
        in_specs=[
            pl.BlockSpec((_PBLK, _H), lambda i: (i, 0)),
            pl.BlockSpec((_PBLK, _H), lambda i: (i, 0)),
            pl.BlockSpec((_H, _H), lambda i: (0, 0)),
            pl.BlockSpec((_H, _H), lambda i: (0, 0)),
            pl.BlockSpec((1, _H), lambda i: (0, 0)),
            pl.BlockSpec((_H, 64), lambda i: (0, 0)),
            pl.BlockSpec((1, 64), lambda i: (0, 0)),
            pl.BlockSpec((1, 64), lambda i: (0, 0)),
            pl.BlockSpec((1, 1), lambda i: (0, 0)),
        ],
        out_specs=pl.BlockSpec((_PBLK, 1), lambda i: (i, 0)),
        out_shape=jax.ShapeDtypeStruct((_PP, 1), _f32),
    )(u, v, w1u, w1v, b1, w2, b2, w3r, b3)


def _worker_id():
    return lax.axis_index("s") * _NC + lax.axis_index("c")


def _split(nch, nworkers, wid):
    nbase, extra = nch // nworkers, nch % nworkers
    start = nbase * wid + jnp.minimum(wid, extra)
    count = nbase + (wid < extra).astype(jnp.int32)
    return start, count


def _chunk_range(nch, wid):
    return _split(nch, _NW, wid)


def _row_loop(s, fn):
    start, count = _split(_NRC, _NS, s)

    def body(k, _):
        fn(pl.multiple_of((start + k) * _RCH, _RCH))
        return 0
    lax.fori_loop(0, count, body, 0)


def _deg_count(dst):
    mesh = plsc.VectorSubcoreMesh(core_axis_name="c", subcore_axis_name="s")
    out_type = [jax.ShapeDtypeStruct((_NC, _N, _H), _f32)]
    scratch = [
        pltpu.VMEM((_CH,), jnp.int32),
        pltpu.VMEM((_CH, _H), _f32),
        pltpu.VMEM((_CH, _H), _f32),
        pltpu.VMEM_SHARED((_N, _H), _f32),
    ]

    def body(dst_hbm, deg_out, dst_v, ones_v, zero_v, deg_sh):
        c = lax.axis_index("c")
        s = lax.axis_index("s")
        wid = _worker_id()
        zero16 = jnp.zeros((16,), _f32)
        one16 = jnp.ones((16,), _f32)

        def fill(i, _):
            for j in range(_H // 16):
                ones_v[i, pl.ds(j * 16, 16)] = one16
                zero_v[i, pl.ds(j * 16, 16)] = zero16
            return 0
        lax.fori_loop(0, _CH, fill, 0)

        _row_loop(s, lambda r0: pltpu.sync_copy(
            zero_v.at[pl.ds(0, _RCH), :], deg_sh.at[pl.ds(r0, _RCH), :]))
        plsc.subcore_barrier()

        start, count = _chunk_range(_E // _CH, wid)

        def chunk(i, _):
            e0 = pl.multiple_of((start + i) * _CH, _CH)
            pltpu.sync_copy(dst_hbm.at[pl.ds(e0, _CH)], dst_v)
            pltpu.sync_copy(ones_v, deg_sh.at[dst_v], add=True)
            return 0
        lax.fori_loop(0, count, chunk, 0)
        plsc.subcore_barrier()

        _row_loop(s, lambda r0: pltpu.sync_copy(
            deg_sh.at[pl.ds(r0, _RCH), :], deg_out.at[c, pl.ds(r0, _RCH), :]))

    return pl.kernel(body, mesh=mesh, out_type=out_type,
                     scratch_types=scratch)(dst)[0]


def _edge_agg(m, src, dst):
    mesh = plsc.VectorSubcoreMesh(core_axis_name="c", subcore_axis_name="s")
    out_type = [jax.ShapeDtypeStruct((_NC, _N, _H), _f32)]
    scratch = [
        pltpu.VMEM((_CH, _H), _f32),
        pltpu.VMEM((_CH,), jnp.int32),
        pltpu.VMEM((_CH,), jnp.int32),
        pltpu.VMEM_SHARED((_N, _H), _f32),
    ]

    def body(m_hbm, src_hbm, dst_hbm, agg_out, rows_v, src_v, dst_v, agg_sh):
        c = lax.axis_index("c")
        s = lax.axis_index("s")
        wid = _worker_id()
        zero16 = jnp.zeros((16,), _f32)

        def fill(i, _):
            for j in range(_H // 16):
                rows_v[i, pl.ds(j * 16, 16)] = zero16
            return 0
        lax.fori_loop(0, _CH, fill, 0)

        _row_loop(s, lambda r0: pltpu.sync_copy(
            rows_v.at[pl.ds(0, _RCH), :], agg_sh.at[pl.ds(r0, _RCH), :]))
        plsc.subcore_barrier()

        start, count = _chunk_range(_E // _CH, wid)

        def chunk(i, _):
            e0 = pl.multiple_of((start + i) * _CH, _CH)
            pltpu.sync_copy(src_hbm.at[pl.ds(e0, _CH)], src_v)
            pltpu.sync_copy(dst_hbm.at[pl.ds(e0, _CH)], dst_v)
            pltpu.sync_copy(m_hbm.at[src_v], rows_v)
            pltpu.sync_copy(rows_v, agg_sh.at[dst_v], add=True)
            return 0
        lax.fori_loop(0, count, chunk, 0)
        plsc.subcore_barrier()

        _row_loop(s, lambda r0: pltpu.sync_copy(
            agg_sh.at[pl.ds(r0, _RCH), :], agg_out.at[c, pl.ds(r0, _RCH), :]))

    return pl.kernel(body, mesh=mesh, out_type=out_type,
                     scratch_types=scratch)(m, src, dst)[0]


def _pair_gather(h, pu, pv):
    mesh = plsc.VectorSubcoreMesh(core_axis_name="c", subcore_axis_name="s")
    out_type = [jax.ShapeDtypeStruct((_PP, _H), _f32)] * 2
    scratch = [
        pltpu.VMEM((_CH,), jnp.int32),
        pltpu.VMEM((_CH, _H), _f32),
    ]

    def body(h_hbm, pu_hbm, pv_hbm, u_out, v_out, idx_v, rows_v):
        wid = _worker_id()
        start, count = _chunk_range(_PP // _CH, wid)

        def chunk(i, _):
            e0 = pl.multiple_of((start + i) * _CH, _CH)
            pltpu.sync_copy(pu_hbm.at[pl.ds(e0, _CH)], idx_v)
            pltpu.sync_copy(h_hbm.at[idx_v], rows_v)
            pltpu.sync_copy(rows_v, u_out.at[pl.ds(e0, _CH), :])
            pltpu.sync_copy(pv_hbm.at[pl.ds(e0, _CH)], idx_v)
            pltpu.sync_copy(h_hbm.at[idx_v], rows_v)
            pltpu.sync_copy(rows_v, v_out.at[pl.ds(e0, _CH), :])
            return 0
        lax.fori_loop(0, count, chunk, 0)

    return pl.kernel(body, mesh=mesh, out_type=out_type,
                     scratch_types=scratch)(h, pu, pv)


def kernel(x, edge_index, edge_type, edge_pairs, W_rel0, W_root0, b0,
           W_rel1, W_root1, b1, Wl1, bl1, Wl2, bl2, Wl3, bl3):
    del edge_type
    src = edge_index[0]
    dst = edge_index[1]
    pp = jnp.zeros((_PP, 2), jnp.int32).at[:_P].set(edge_pairs)
    pu = pp[:, 0]
    pv = pp[:, 1]

    deg = _deg_count(dst)
    m0, r0 = _pre(x, W_rel0[0], W_root0, b0.reshape(1, _H))
    agg0 = _edge_agg(m0, src, dst)
    m1, r1 = _mid(r0, agg0, deg, W_rel1[0], W_root1, b1.reshape(1, _H))
    agg1 = _edge_agg(m1, src, dst)
    h2 = _fin(r1, agg1, deg)
    u, v = _pair_gather(h2, pu, pv)
    z = _mlp(u, v, Wl1[:_H], Wl1[_H:], bl1.reshape(1, _H),
             Wl2, bl2.reshape(1, 64), Wl3.reshape(1, 64), bl3.reshape(1, 1))
    return z[:_P, 0]


# speedup vs baseline: 6.9168x; 1.4131x over previous
import jax
import jax.numpy as jnp
from jax import lax
from jax.experimental import pallas as pl
from jax.experimental.pallas import tpu as pltpu
from jax.experimental.pallas import tpu_sc as plsc

_N = 10000
_E = 320000
_D = 128
_H = 128
_P = 10000
_PP = 10240
_CH = 128
_NC = 2
_NS = 16
_NW = _NC * _NS
_RCH = 16
_NRC = _N // _RCH
_TROWS = 80
_GRP = 4
_BLK = 1000
_PBLK = 1024

_f32 = jnp.float32


def _pre_body(x_ref, wa_ref, wb_ref, b_ref, m_ref, r_ref):
    xb = x_ref[...]
    m_ref[...] = jnp.dot(xb, wa_ref[...], preferred_element_type=_f32)
    r_ref[...] = jnp.dot(xb, wb_ref[...], preferred_element_type=_f32) + b_ref[...]


def _pre(x, wrel, wroot, b):
    return pl.pallas_call(
        _pre_body,
        grid=(_N // _BLK,),
        in_specs=[
            pl.BlockSpec((_BLK, _D), lambda i: (i, 0)),
            pl.BlockSpec((_D, _H), lambda i: (0, 0)),
            pl.BlockSpec((_D, _H), lambda i: (0, 0)),
            pl.BlockSpec((1, _H), lambda i: (0, 0)),
        ],
        out_specs=[pl.BlockSpec((_BLK, _H), lambda i: (i, 0))] * 2,
        out_shape=[jax.ShapeDtypeStruct((_N, _H), _f32)] * 2,
    )(x, wrel, wroot, b)


def _norm_h(r_ref, agg_ref, deg_ref):
    agg = agg_ref[0] + agg_ref[1]
    deg = deg_ref[0, :, 0:1] + deg_ref[1, :, 0:1]
    inv = 1.0 / jnp.maximum(deg, 1.0)
    return jnp.maximum(r_ref[...] + agg * inv, 0.0)


def _mid_body(r_ref, agg_ref, deg_ref, wa_ref, wb_ref, b_ref, m_ref, rn_ref):
    h = _norm_h(r_ref, agg_ref, deg_ref)
    m_ref[...] = jnp.dot(h, wa_ref[...], preferred_element_type=_f32)
    rn_ref[...] = jnp.dot(h, wb_ref[...], preferred_element_type=_f32) + b_ref[...]


def _mid(r0, agg, deg, wrel, wroot, b):
    return pl.pallas_call(
        _mid_body,
        grid=(_N // _BLK,),
        in_specs=[
            pl.BlockSpec((_BLK, _H), lambda i: (i, 0)),
            pl.BlockSpec((_NC, _BLK, _H), lambda i: (0, i, 0)),
            pl.BlockSpec((_NC, _BLK, _H), lambda i: (0, i, 0)),
            pl.BlockSpec((_H, _H), lambda i: (0, 0)),
            pl.BlockSpec((_H, _H), lambda i: (0, 0)),
            pl.BlockSpec((1, _H), lambda i: (0, 0)),
        ],
        out_specs=[pl.BlockSpec((_BLK, _H), lambda i: (i, 0))] * 2,
        out_shape=[jax.ShapeDtypeStruct((_N, _H), _f32)] * 2,
    )(r0, agg, deg, wrel, wroot, b)


def _fin_body(r_ref, agg_ref, deg_ref, h_ref):
    h_ref[...] = _norm_h(r_ref, agg_ref, deg_ref)


def _fin(r1, agg, deg):
    return pl.pallas_call(
        _fin_body,
        grid=(_N // _BLK,),
        in_specs=[
            pl.BlockSpec((_BLK, _H), lambda i: (i, 0)),
            pl.BlockSpec((_NC, _BLK, _H), lambda i: (0, i, 0)),
            pl.BlockSpec((_NC, _BLK, _H), lambda i: (0, i, 0)),
        ],
        out_specs=pl.BlockSpec((_BLK, _H), lambda i: (i, 0)),
        out_shape=jax.ShapeDtypeStruct((_N, _H), _f32),
    )(r1, agg, deg)


def _mlp_body(u_ref, v_ref, w1u_ref, w1v_ref, b1_ref, w2_ref, b2_ref,
              w3_ref, b3_ref, o_ref):
    z = jnp.maximum(
        jnp.dot(u_ref[...], w1u_ref[...], preferred_element_type=_f32)
        + jnp.dot(v_ref[...], w1v_ref[...], preferred_element_type=_f32)
        + b1_ref[...], 0.0)
    z = jnp.maximum(jnp.dot(z, w2_ref[...], preferred_element_type=_f32)
                    + b2_ref[...], 0.0)
    t = jnp.sum(z * w3_ref[...], axis=1, keepdims=True) + b3_ref[...]
    o_ref[...] = 1.0 / (1.0 + jnp.exp(-t))


def _mlp(u, v, w1u, w1v, b1, w2, b2, w3r, b3):
    return pl.pallas_call(
        _mlp_body,
        grid=(_PP // _PBLK,),
        in_specs=[
            pl.BlockSpec((_PBLK, _H), lambda i: (i, 0)),
            pl.BlockSpec((_PBLK, _H), lambda i: (i, 0)),
            pl.BlockSpec((_H, _H), lambda i: (0, 0)),
            pl.BlockSpec((_H, _H), lambda i: (0, 0)),
            pl.BlockSpec((1, _H), lambda i: (0, 0)),
            pl.BlockSpec((_H, 64), lambda i: (0, 0)),
            pl.BlockSpec((1, 64), lambda i: (0, 0)),
            pl.BlockSpec((1, 64), lambda i: (0, 0)),
            pl.BlockSpec((1, 1), lambda i: (0, 0)),
        ],
        out_specs=pl.BlockSpec((_PBLK, 1), lambda i: (i, 0)),
        out_shape=jax.ShapeDtypeStruct((_PP, 1), _f32),
    )(u, v, w1u, w1v, b1, w2, b2, w3r, b3)


def _worker_id():
    return lax.axis_index("s") * _NC + lax.axis_index("c")


def _split(nch, nworkers, wid):
    nbase, extra = nch // nworkers, nch % nworkers
    start = nbase * wid + jnp.minimum(wid, extra)
    count = nbase + (wid < extra).astype(jnp.int32)
    return start, count


def _chunk_range(nch, wid):
    return _split(nch, _NW, wid)


def _row_loop(s, fn):
    start, count = _split(_NRC, _NS, s)

    def body(k, _):
        fn(pl.multiple_of((start + k) * _RCH, _RCH))
        return 0
    lax.fori_loop(0, count, body, 0)


def _tile_chunks(wid):
    base = pl.multiple_of(wid * _TROWS, _TROWS)
    count = jnp.minimum(jnp.maximum(_E // _CH - wid * _TROWS, 0), _TROWS)
    return base, count


def _deg_count(dst2):
    mesh = plsc.VectorSubcoreMesh(core_axis_name="c", subcore_axis_name="s")
    out_type = [jax.ShapeDtypeStruct((_NC, _N, _H), _f32)]
    scratch = [
        pltpu.VMEM((_TROWS, _CH), jnp.int32),
        pltpu.VMEM((_CH, _H), _f32),
        pltpu.VMEM((_CH, _H), _f32),
        pltpu.VMEM_SHARED((_N, _H), _f32),
        pltpu.SemaphoreType.DMA((_GRP,)),
    ]

    def body(dst_hbm, deg_out, idx_d, ones_v, zero_v, deg_sh, ssem):
        c = lax.axis_index("c")
        s = lax.axis_index("s")
        wid = _worker_id()
        zero16 = jnp.zeros((16,), _f32)
        one16 = jnp.ones((16,), _f32)

        def fill(i, _):
            for j in range(_H // 16):
                ones_v[i, pl.ds(j * 16, 16)] = one16
                zero_v[i, pl.ds(j * 16, 16)] = zero16
            return 0
        lax.fori_loop(0, _CH, fill, 0)

        _row_loop(s, lambda r0: pltpu.sync_copy(
            zero_v.at[pl.ds(0, _RCH), :], deg_sh.at[pl.ds(r0, _RCH), :]))
        plsc.subcore_barrier()

        base, count = _tile_chunks(wid)
        pltpu.sync_copy(dst_hbm.at[pl.ds(base, _TROWS), :], idx_d)

        def group(p, _):
            hs = [pltpu.async_copy(ones_v, deg_sh.at[idx_d.at[p * _GRP + b]],
                                   ssem.at[b], add=True)
                  for b in range(_GRP)]
            for h in hs:
                h.wait()
            return 0
        lax.fori_loop(0, count // _GRP, group, 0)
        plsc.subcore_barrier()

        _row_loop(s, lambda r0: pltpu.sync_copy(
            deg_sh.at[pl.ds(r0, _RCH), :], deg_out.at[c, pl.ds(r0, _RCH), :]))

    return pl.kernel(body, mesh=mesh, out_type=out_type,
                     scratch_types=scratch)(dst2)[0]


def _edge_agg(m, src2, dst2):
    mesh = plsc.VectorSubcoreMesh(core_axis_name="c", subcore_axis_name="s")
    out_type = [jax.ShapeDtypeStruct((_NC, _N, _H), _f32)]
    hrows = _TROWS // 2
    scratch = [
        pltpu.VMEM((2, _CH, _H), _f32),
        pltpu.VMEM((hrows, _CH), jnp.int32),
        pltpu.VMEM((hrows, _CH), jnp.int32),
        pltpu.VMEM_SHARED((_N, _H), _f32),
        pltpu.SemaphoreType.DMA((2,)),
        pltpu.SemaphoreType.DMA((2,)),
    ]

    def body(m_hbm, src_hbm, dst_hbm, agg_out, rows_v, idx_s, idx_d, agg_sh,
             gsem, ssem):
        c = lax.axis_index("c")
        s = lax.axis_index("s")
        wid = _worker_id()
        zero16 = jnp.zeros((16,), _f32)

        def fill(i, _):
            for j in range(_H // 16):
                rows_v[0, i, pl.ds(j * 16, 16)] = zero16
            return 0
        lax.fori_loop(0, _CH, fill, 0)

        _row_loop(s, lambda r0: pltpu.sync_copy(
            rows_v.at[0, pl.ds(0, _RCH), :], agg_sh.at[pl.ds(r0, _RCH), :]))
        plsc.subcore_barrier()

        base, count = _tile_chunks(wid)
        for hh in range(2):
            bh = pl.multiple_of(base + hh * hrows, 8)
            cnt = jnp.minimum(jnp.maximum(count - hh * hrows, 0), hrows)
            pltpu.sync_copy(src_hbm.at[pl.ds(bh, hrows), :], idx_s)
            pltpu.sync_copy(dst_hbm.at[pl.ds(bh, hrows), :], idx_d)

            def group(p, _):
                gs = [pltpu.async_copy(m_hbm.at[idx_s.at[p * 2 + b]],
                                       rows_v.at[b], gsem.at[b])
                      for b in range(2)]
                ss = []
                for b in range(2):
                    gs[b].wait()
                    ss.append(pltpu.async_copy(
                        rows_v.at[b], agg_sh.at[idx_d.at[p * 2 + b]],
                        ssem.at[b], add=True))
                for h in ss:
                    h.wait()
                return 0
            lax.fori_loop(0, cnt // 2, group, 0)
        plsc.subcore_barrier()

        _row_loop(s, lambda r0: pltpu.sync_copy(
            agg_sh.at[pl.ds(r0, _RCH), :], agg_out.at[c, pl.ds(r0, _RCH), :]))

    return pl.kernel(body, mesh=mesh, out_type=out_type,
                     scratch_types=scratch)(m, src2, dst2)[0]


def _pair_gather(h, pu2, pv2):
    mesh = plsc.VectorSubcoreMesh(core_axis_name="c", subcore_axis_name="s")
    out_type = [jax.ShapeDtypeStruct((_PP, _H), _f32)] * 2
    nch = _PP // _CH
    scratch = [
        pltpu.VMEM((nch, _CH), jnp.int32),
        pltpu.VMEM((nch, _CH), jnp.int32),
        pltpu.VMEM((_CH, _H), _f32),
        pltpu.VMEM((_CH, _H), _f32),
        pltpu.SemaphoreType.DMA((2,)),
    ]

    def body(h_hbm, pu_hbm, pv_hbm, u_out, v_out, idx_u, idx_v, ru_v, rv_v,
             gsem):
        wid = _worker_id()
        pltpu.sync_copy(pu_hbm, idx_u)
        pltpu.sync_copy(pv_hbm, idx_v)
        count = 2 + (wid < (nch - 2 * _NW)).astype(jnp.int32)

        def chunk(g, _):
            j = wid + _NW * g
            hu = pltpu.async_copy(h_hbm.at[idx_u.at[j]], ru_v, gsem.at[0])
            hv = pltpu.async_copy(h_hbm.at[idx_v.at[j]], rv_v, gsem.at[1])
            e0 = pl.multiple_of(j * _CH, _CH)
            hu.wait()
            pltpu.sync_copy(ru_v, u_out.at[pl.ds(e0, _CH), :])
            hv.wait()
            pltpu.sync_copy(rv_v, v_out.at[pl.ds(e0, _CH), :])
            return 0
        lax.fori_loop(0, count, chunk, 0)

    return pl.kernel(body, mesh=mesh, out_type=out_type,
                     scratch_types=scratch)(h, pu2, pv2)


def kernel(x, edge_index, edge_type, edge_pairs, W_rel0, W_root0, b0,
           W_rel1, W_root1, b1, Wl1, bl1, Wl2, bl2, Wl3, bl3):
    del edge_type
    npad = _NW * _TROWS * _CH - _E
    zpad = jnp.zeros((npad,), jnp.int32)
    src2 = jnp.concatenate([edge_index[0], zpad]).reshape(_NW * _TROWS, _CH)
    dst2 = jnp.concatenate([edge_index[1], zpad]).reshape(_NW * _TROWS, _CH)
    pp = jnp.zeros((_PP, 2), jnp.int32).at[:_P].set(edge_pairs)
    pu2 = pp[:, 0].reshape(_PP // _CH, _CH)
    pv2 = pp[:, 1].reshape(_PP // _CH, _CH)

    deg = _deg_count(dst2)
    m0, r0 = _pre(x, W_rel0[0], W_root0, b0.reshape(1, _H))
    agg0 = _edge_agg(m0, src2, dst2)
    m1, r1 = _mid(r0, agg0, deg, W_rel1[0], W_root1, b1.reshape(1, _H))
    agg1 = _edge_agg(m1, src2, dst2)
    h2 = _fin(r1, agg1, deg)
    u, v = _pair_gather(h2, pu2, pv2)
    z = _mlp(u, v, Wl1[:_H], Wl1[_H:], bl1.reshape(1, _H),
             Wl2, bl2.reshape(1, 64), Wl3.reshape(1, 64), bl3.reshape(1, 1))
    return z[:_P, 0]


# speedup vs baseline: 6.9804x; 1.0092x over previous
import jax
import jax.numpy as jnp
from jax import lax
from jax.experimental import pallas as pl
from jax.experimental.pallas import tpu as pltpu
from jax.experimental.pallas import tpu_sc as plsc

_N = 10000
_E = 320000
_D = 128
_H = 128
_P = 10000
_PP = 10240
_CH = 128
_NC = 2
_NS = 16
_NW = _NC * _NS
_RCH = 16
_NRC = _N // _RCH
_TROWS = 80
_GRP = 4
_BLK = 1000
_PBLK = 1024

_f32 = jnp.float32


def _pre_body(x_ref, wa_ref, wb_ref, b_ref, m_ref, r_ref):
    xb = x_ref[...]
    m_ref[...] = jnp.dot(xb, wa_ref[...], preferred_element_type=_f32)
    r_ref[...] = jnp.dot(xb, wb_ref[...], preferred_element_type=_f32) + b_ref[...]


def _pre(x, wrel, wroot, b):
    return pl.pallas_call(
        _pre_body,
        grid=(_N // _BLK,),
        in_specs=[
            pl.BlockSpec((_BLK, _D), lambda i: (i, 0)),
            pl.BlockSpec((_D, _H), lambda i: (0, 0)),
            pl.BlockSpec((_D, _H), lambda i: (0, 0)),
            pl.BlockSpec((1, _H), lambda i: (0, 0)),
        ],
        out_specs=[pl.BlockSpec((_BLK, _H), lambda i: (i, 0))] * 2,
        out_shape=[jax.ShapeDtypeStruct((_N, _H), _f32)] * 2,
    )(x, wrel, wroot, b)


def _norm_h(r_ref, agg_ref, deg_ref):
    agg = agg_ref[0] + agg_ref[1]
    deg = deg_ref[0, :, 0:1] + deg_ref[1, :, 0:1]
    inv = 1.0 / jnp.maximum(deg, 1.0)
    return jnp.maximum(r_ref[...] + agg * inv, 0.0)


def _mid_body(r_ref, agg_ref, deg_ref, wa_ref, wb_ref, b_ref, m_ref, rn_ref):
    h = _norm_h(r_ref, agg_ref, deg_ref)
    m_ref[...] = jnp.dot(h, wa_ref[...], preferred_element_type=_f32)
    rn_ref[...] = jnp.dot(h, wb_ref[...], preferred_element_type=_f32) + b_ref[...]


def _mid(r0, agg, deg, wrel, wroot, b):
    return pl.pallas_call(
        _mid_body,
        grid=(_N // _BLK,),
        in_specs=[
            pl.BlockSpec((_BLK, _H), lambda i: (i, 0)),
            pl.BlockSpec((_NC, _BLK, _H), lambda i: (0, i, 0)),
            pl.BlockSpec((_NC, _BLK, _H), lambda i: (0, i, 0)),
            pl.BlockSpec((_H, _H), lambda i: (0, 0)),
            pl.BlockSpec((_H, _H), lambda i: (0, 0)),
            pl.BlockSpec((1, _H), lambda i: (0, 0)),
        ],
        out_specs=[pl.BlockSpec((_BLK, _H), lambda i: (i, 0))] * 2,
        out_shape=[jax.ShapeDtypeStruct((_N, _H), _f32)] * 2,
    )(r0, agg, deg, wrel, wroot, b)


def _fin_body(r_ref, agg_ref, deg_ref, h_ref):
    h_ref[...] = _norm_h(r_ref, agg_ref, deg_ref)


def _fin(r1, agg, deg):
    return pl.pallas_call(
        _fin_body,
        grid=(_N // _BLK,),
        in_specs=[
            pl.BlockSpec((_BLK, _H), lambda i: (i, 0)),
            pl.BlockSpec((_NC, _BLK, _H), lambda i: (0, i, 0)),
            pl.BlockSpec((_NC, _BLK, _H), lambda i: (0, i, 0)),
        ],
        out_specs=pl.BlockSpec((_BLK, _H), lambda i: (i, 0)),
        out_shape=jax.ShapeDtypeStruct((_N, _H), _f32),
    )(r1, agg, deg)


def _mlp_body(u_ref, v_ref, w1u_ref, w1v_ref, b1_ref, w2_ref, b2_ref,
              w3_ref, b3_ref, o_ref):
    z = jnp.maximum(
        jnp.dot(u_ref[...], w1u_ref[...], preferred_element_type=_f32)
        + jnp.dot(v_ref[...], w1v_ref[...], preferred_element_type=_f32)
        + b1_ref[...], 0.0)
    z = jnp.maximum(jnp.dot(z, w2_ref[...], preferred_element_type=_f32)
                    + b2_ref[...], 0.0)
    t = jnp.sum(z * w3_ref[...], axis=1, keepdims=True) + b3_ref[...]
    o_ref[...] = 1.0 / (1.0 + jnp.exp(-t))


def _mlp(u, v, w1u, w1v, b1, w2, b2, w3r, b3):
    return pl.pallas_call(
        _mlp_body,
        grid=(_PP // _PBLK,),
        in_specs=[
            pl.BlockSpec((_PBLK, _H), lambda i: (i, 0)),
            pl.BlockSpec((_PBLK, _H), lambda i: (i, 0)),
            pl.BlockSpec((_H, _H), lambda i: (0, 0)),
            pl.BlockSpec((_H, _H), lambda i: (0, 0)),
            pl.BlockSpec((1, _H), lambda i: (0, 0)),
            pl.BlockSpec((_H, 64), lambda i: (0, 0)),
            pl.BlockSpec((1, 64), lambda i: (0, 0)),
            pl.BlockSpec((1, 64), lambda i: (0, 0)),
            pl.BlockSpec((1, 1), lambda i: (0, 0)),
        ],
        out_specs=pl.BlockSpec((_PBLK, 1), lambda i: (i, 0)),
        out_shape=jax.ShapeDtypeStruct((_PP, 1), _f32),
    )(u, v, w1u, w1v, b1, w2, b2, w3r, b3)


def _worker_id():
    return lax.axis_index("s") * _NC + lax.axis_index("c")


def _split(nch, nworkers, wid):
    nbase, extra = nch // nworkers, nch % nworkers
    start = nbase * wid + jnp.minimum(wid, extra)
    count = nbase + (wid < extra).astype(jnp.int32)
    return start, count


def _chunk_range(nch, wid):
    return _split(nch, _NW, wid)


def _row_loop(s, fn):
    start, count = _split(_NRC, _NS, s)

    def body(k, _):
        fn(pl.multiple_of((start + k) * _RCH, _RCH))
        return 0
    lax.fori_loop(0, count, body, 0)


def _tile_chunks(wid):
    base = pl.multiple_of(wid * _TROWS, _TROWS)
    count = jnp.minimum(jnp.maximum(_E // _CH - wid * _TROWS, 0), _TROWS)
    return base, count


def _deg_count(dst2):
    mesh = plsc.VectorSubcoreMesh(core_axis_name="c", subcore_axis_name="s")
    out_type = [jax.ShapeDtypeStruct((_NC, _N, _H), _f32)]
    scratch = [
        pltpu.VMEM((_TROWS, _CH), jnp.int32),
        pltpu.VMEM((_CH, _H), _f32),
        pltpu.VMEM((_CH, _H), _f32),
        pltpu.VMEM_SHARED((_N, _H), _f32),
        pltpu.SemaphoreType.DMA((_GRP,)),
    ]

    def body(dst_hbm, deg_out, idx_d, ones_v, zero_v, deg_sh, ssem):
        c = lax.axis_index("c")
        s = lax.axis_index("s")
        wid = _worker_id()
        zero16 = jnp.zeros((16,), _f32)
        one16 = jnp.ones((16,), _f32)

        def fill(i, _):
            for j in range(_H // 16):
                ones_v[i, pl.ds(j * 16, 16)] = one16
                zero_v[i, pl.ds(j * 16, 16)] = zero16
            return 0
        lax.fori_loop(0, _CH, fill, 0)

        _row_loop(s, lambda r0: pltpu.sync_copy(
            zero_v.at[pl.ds(0, _RCH), :], deg_sh.at[pl.ds(r0, _RCH), :]))
        plsc.subcore_barrier()

        base, count = _tile_chunks(wid)
        pltpu.sync_copy(dst_hbm.at[pl.ds(base, _TROWS), :], idx_d)

        def group(p, _):
            hs = [pltpu.async_copy(ones_v, deg_sh.at[idx_d.at[p * _GRP + b]],
                                   ssem.at[b], add=True)
                  for b in range(_GRP)]
            for h in hs:
                h.wait()
            return 0
        lax.fori_loop(0, count // _GRP, group, 0)
        plsc.subcore_barrier()

        _row_loop(s, lambda r0: pltpu.sync_copy(
            deg_sh.at[pl.ds(r0, _RCH), :], deg_out.at[c, pl.ds(r0, _RCH), :]))

    return pl.kernel(body, mesh=mesh, out_type=out_type,
                     scratch_types=scratch)(dst2)[0]


def _edge_agg(m, src2, dst2):
    mesh = plsc.VectorSubcoreMesh(core_axis_name="c", subcore_axis_name="s")
    out_type = [jax.ShapeDtypeStruct((_NC, _N, _H), _f32)]
    hrows = _TROWS // 2
    scratch = [
        pltpu.VMEM((2, _CH, _H), _f32),
        pltpu.VMEM((hrows, _CH), jnp.int32),
        pltpu.VMEM((hrows, _CH), jnp.int32),
        pltpu.VMEM_SHARED((_N, _H), _f32),
        pltpu.SemaphoreType.DMA((2,)),
        pltpu.SemaphoreType.DMA((2,)),
    ]

    def body(m_hbm, src_hbm, dst_hbm, agg_out, rows_v, idx_s, idx_d, agg_sh,
             gsem, ssem):
        c = lax.axis_index("c")
        s = lax.axis_index("s")
        wid = _worker_id()
        zero16 = jnp.zeros((16,), _f32)

        def fill(i, _):
            for j in range(_H // 16):
                rows_v[0, i, pl.ds(j * 16, 16)] = zero16
            return 0
        lax.fori_loop(0, _CH, fill, 0)

        _row_loop(s, lambda r0: pltpu.sync_copy(
            rows_v.at[0, pl.ds(0, _RCH), :], agg_sh.at[pl.ds(r0, _RCH), :]))
        plsc.subcore_barrier()

        base, count = _tile_chunks(wid)

        def drain_scatter(b):
            pltpu.make_async_copy(m_hbm.at[pl.ds(0, _CH), :], rows_v.at[b],
                                  ssem.at[b]).wait()

        for hh in range(2):
            bh = pl.multiple_of(base + hh * hrows, 8)
            cnt = jnp.minimum(jnp.maximum(count - hh * hrows, 0), hrows)
            pltpu.sync_copy(src_hbm.at[pl.ds(bh, hrows), :], idx_s)
            pltpu.sync_copy(dst_hbm.at[pl.ds(bh, hrows), :], idx_d)

            def group(p, _):
                gs = []
                for b in range(2):
                    @pl.when(p > 0)
                    def _():
                        drain_scatter(b)
                    gs.append(pltpu.async_copy(m_hbm.at[idx_s.at[p * 2 + b]],
                                               rows_v.at[b], gsem.at[b]))
                for b in range(2):
                    gs[b].wait()
                    pltpu.async_copy(rows_v.at[b],
                                     agg_sh.at[idx_d.at[p * 2 + b]],
                                     ssem.at[b], add=True)
                return 0
            lax.fori_loop(0, cnt // 2, group, 0)
            @pl.when(cnt > 0)
            def _():
                for b in range(2):
                    drain_scatter(b)
        plsc.subcore_barrier()

        _row_loop(s, lambda r0: pltpu.sync_copy(
            agg_sh.at[pl.ds(r0, _RCH), :], agg_out.at[c, pl.ds(r0, _RCH), :]))

    return pl.kernel(body, mesh=mesh, out_type=out_type,
                     scratch_types=scratch)(m, src2, dst2)[0]


def _pair_gather(h, pu2, pv2):
    mesh = plsc.VectorSubcoreMesh(core_axis_name="c", subcore_axis_name="s")
    out_type = [jax.ShapeDtypeStruct((_PP, _H), _f32)] * 2
    nch = _PP // _CH
    scratch = [
        pltpu.VMEM((nch, _CH), jnp.int32),
        pltpu.VMEM((nch, _CH), jnp.int32),
        pltpu.VMEM((_CH, _H), _f32),
        pltpu.VMEM((_CH, _H), _f32),
        pltpu.SemaphoreType.DMA((2,)),
    ]

    def body(h_hbm, pu_hbm, pv_hbm, u_out, v_out, idx_u, idx_v, ru_v, rv_v,
             gsem):
        wid = _worker_id()
        pltpu.sync_copy(pu_hbm, idx_u)
        pltpu.sync_copy(pv_hbm, idx_v)
        count = 2 + (wid < (nch - 2 * _NW)).astype(jnp.int32)

        def chunk(g, _):
            j = wid + _NW * g
            hu = pltpu.async_copy(h_hbm.at[idx_u.at[j]], ru_v, gsem.at[0])
            hv = pltpu.async_copy(h_hbm.at[idx_v.at[j]], rv_v, gsem.at[1])
            e0 = pl.multiple_of(j * _CH, _CH)
            hu.wait()
            pltpu.sync_copy(ru_v, u_out.at[pl.ds(e0, _CH), :])
            hv.wait()
            pltpu.sync_copy(rv_v, v_out.at[pl.ds(e0, _CH), :])
            return 0
        lax.fori_loop(0, count, chunk, 0)

    return pl.kernel(body, mesh=mesh, out_type=out_type,
                     scratch_types=scratch)(h, pu2, pv2)


def kernel(x, edge_index, edge_type, edge_pairs, W_rel0, W_root0, b0,
           W_rel1, W_root1, b1, Wl1, bl1, Wl2, bl2, Wl3, bl3):
    del edge_type
    npad = _NW * _TROWS * _CH - _E
    zpad = jnp.zeros((npad,), jnp.int32)
    src2 = jnp.concatenate([edge_index[0], zpad]).reshape(_NW * _TROWS, _CH)
    dst2 = jnp.concatenate([edge_index[1], zpad]).reshape(_NW * _TROWS, _CH)
    pp = jnp.zeros((_PP, 2), jnp.int32).at[:_P].set(edge_pairs)
    pu2 = pp[:, 0].reshape(_PP // _CH, _CH)
    pv2 = pp[:, 1].reshape(_PP // _CH, _CH)

    deg = _deg_count(dst2)
    m0, r0 = _pre(x, W_rel0[0], W_root0, b0.reshape(1, _H))
    agg0 = _edge_agg(m0, src2, dst2)
    m1, r1 = _mid(r0, agg0, deg, W_rel1[0], W_root1, b1.reshape(1, _H))
    agg1 = _edge_agg(m1, src2, dst2)
    h2 = _fin(r1, agg1, deg)
    u, v = _pair_gather(h2, pu2, pv2)
    z = _mlp(u, v, Wl1[:_H], Wl1[_H:], bl1.reshape(1, _H),
             Wl2, bl2.reshape(1, 64), Wl3.reshape(1, 64), bl3.reshape(1, 1))
    return z[:_P, 0]


# speedup vs baseline: 6.9934x; 1.0019x over previous
import jax
import jax.numpy as jnp
from jax import lax
from jax.experimental import pallas as pl
from jax.experimental.pallas import tpu as pltpu
from jax.experimental.pallas import tpu_sc as plsc

_N = 10000
_E = 320000
_D = 128
_H = 128
_P = 10000
_PP = 10240
_CH = 128
_NC = 2
_NS = 16
_NW = _NC * _NS
_RCH = 16
_NRC = _N // _RCH
_TROWS = 80
_GRP = 4
_DGRP = 10
_BLK = 1000
_PBLK = 1024

_f32 = jnp.float32


def _pre_body(x_ref, wa_ref, wb_ref, b_ref, m_ref, r_ref):
    xb = x_ref[...]
    m_ref[...] = jnp.dot(xb, wa_ref[...], preferred_element_type=_f32)
    r_ref[...] = jnp.dot(xb, wb_ref[...], preferred_element_type=_f32) + b_ref[...]


def _pre(x, wrel, wroot, b):
    return pl.pallas_call(
        _pre_body,
        grid=(_N // _BLK,),
        in_specs=[
            pl.BlockSpec((_BLK, _D), lambda i: (i, 0)),
            pl.BlockSpec((_D, _H), lambda i: (0, 0)),
            pl.BlockSpec((_D, _H), lambda i: (0, 0)),
            pl.BlockSpec((1, _H), lambda i: (0, 0)),
        ],
        out_specs=[pl.BlockSpec((_BLK, _H), lambda i: (i, 0))] * 2,
        out_shape=[jax.ShapeDtypeStruct((_N, _H), _f32)] * 2,
    )(x, wrel, wroot, b)


def _norm_h(r_ref, agg_ref, deg_ref):
    agg = agg_ref[0] + agg_ref[1]
    deg = deg_ref[0, :, 0:1] + deg_ref[1, :, 0:1]
    inv = 1.0 / jnp.maximum(deg, 1.0)
    return jnp.maximum(r_ref[...] + agg * inv, 0.0)


def _mid_body(r_ref, agg_ref, deg_ref, wa_ref, wb_ref, b_ref, m_ref, rn_ref):
    h = _norm_h(r_ref, agg_ref, deg_ref)
    m_ref[...] = jnp.dot(h, wa_ref[...], preferred_element_type=_f32)
    rn_ref[...] = jnp.dot(h, wb_ref[...], preferred_element_type=_f32) + b_ref[...]


def _mid(r0, agg, deg, wrel, wroot, b):
    return pl.pallas_call(
        _mid_body,
        grid=(_N // _BLK,),
        in_specs=[
            pl.BlockSpec((_BLK, _H), lambda i: (i, 0)),
            pl.BlockSpec((_NC, _BLK, _H), lambda i: (0, i, 0)),
            pl.BlockSpec((_NC, _BLK, _H), lambda i: (0, i, 0)),
            pl.BlockSpec((_H, _H), lambda i: (0, 0)),
            pl.BlockSpec((_H, _H), lambda i: (0, 0)),
            pl.BlockSpec((1, _H), lambda i: (0, 0)),
        ],
        out_specs=[pl.BlockSpec((_BLK, _H), lambda i: (i, 0))] * 2,
        out_shape=[jax.ShapeDtypeStruct((_N, _H), _f32)] * 2,
    )(r0, agg, deg, wrel, wroot, b)


def _fin_body(r_ref, agg_ref, deg_ref, h_ref):
    h_ref[...] = _norm_h(r_ref, agg_ref, deg_ref)


def _fin(r1, agg, deg):
    return pl.pallas_call(
        _fin_body,
        grid=(_N // _BLK,),
        in_specs=[
            pl.BlockSpec((_BLK, _H), lambda i: (i, 0)),
            pl.BlockSpec((_NC, _BLK, _H), lambda i: (0, i, 0)),
            pl.BlockSpec((_NC, _BLK, _H), lambda i: (0, i, 0)),
        ],
        out_specs=pl.BlockSpec((_BLK, _H), lambda i: (i, 0)),
        out_shape=jax.ShapeDtypeStruct((_N, _H), _f32),
    )(r1, agg, deg)


def _mlp_body(u_ref, v_ref, w1u_ref, w1v_ref, b1_ref, w2_ref, b2_ref,
              w3_ref, b3_ref, o_ref):
    z = jnp.maximum(
        jnp.dot(u_ref[...], w1u_ref[...], preferred_element_type=_f32)
        + jnp.dot(v_ref[...], w1v_ref[...], preferred_element_type=_f32)
        + b1_ref[...], 0.0)
    z = jnp.maximum(jnp.dot(z, w2_ref[...], preferred_element_type=_f32)
                    + b2_ref[...], 0.0)
    t = jnp.sum(z * w3_ref[...], axis=1, keepdims=True) + b3_ref[...]
    o_ref[...] = 1.0 / (1.0 + jnp.exp(-t))


def _mlp(u, v, w1u, w1v, b1, w2, b2, w3r, b3):
    return pl.pallas_call(
        _mlp_body,
        grid=(_PP // _PBLK,),
        in_specs=[
            pl.BlockSpec((_PBLK, _H), lambda i: (i, 0)),
            pl.BlockSpec((_PBLK, _H), lambda i: (i, 0)),
            pl.BlockSpec((_H, _H), lambda i: (0, 0)),
            pl.BlockSpec((_H, _H), lambda i: (0, 0)),
            pl.BlockSpec((1, _H), lambda i: (0, 0)),
            pl.BlockSpec((_H, 64), lambda i: (0, 0)),
            pl.BlockSpec((1, 64), lambda i: (0, 0)),
            pl.BlockSpec((1, 64), lambda i: (0, 0)),
            pl.BlockSpec((1, 1), lambda i: (0, 0)),
        ],
        out_specs=pl.BlockSpec((_PBLK, 1), lambda i: (i, 0)),
        out_shape=jax.ShapeDtypeStruct((_PP, 1), _f32),
    )(u, v, w1u, w1v, b1, w2, b2, w3r, b3)


def _worker_id():
    return lax.axis_index("s") * _NC + lax.axis_index("c")


def _split(nch, nworkers, wid):
    nbase, extra = nch // nworkers, nch % nworkers
    start = nbase * wid + jnp.minimum(wid, extra)
    count = nbase + (wid < extra).astype(jnp.int32)
    return start, count


def _chunk_range(nch, wid):
    return _split(nch, _NW, wid)


def _row_loop(s, fn):
    start, count = _split(_NRC, _NS, s)

    def body(k, _):
        fn(pl.multiple_of((start + k) * _RCH, _RCH))
        return 0
    lax.fori_loop(0, count, body, 0)


def _tile_chunks(wid):
    base = pl.multiple_of(wid * _TROWS, _TROWS)
    count = jnp.minimum(jnp.maximum(_E // _CH - wid * _TROWS, 0), _TROWS)
    return base, count


def _deg_count(dst2):
    mesh = plsc.VectorSubcoreMesh(core_axis_name="c", subcore_axis_name="s")
    out_type = [jax.ShapeDtypeStruct((_NC, _N, _H), _f32)]
    scratch = [
        pltpu.VMEM((_TROWS, _CH), jnp.int32),
        pltpu.VMEM((_CH, _H), _f32),
        pltpu.VMEM((_CH, _H), _f32),
        pltpu.VMEM_SHARED((_N, _H), _f32),
        pltpu.SemaphoreType.DMA((_DGRP,)),
    ]

    def body(dst_hbm, deg_out, idx_d, ones_v, zero_v, deg_sh, ssem):
        c = lax.axis_index("c")
        s = lax.axis_index("s")
        wid = _worker_id()
        zero16 = jnp.zeros((16,), _f32)
        one16 = jnp.ones((16,), _f32)

        def fill(i, _):
            for j in range(_H // 16):
                ones_v[i, pl.ds(j * 16, 16)] = one16
                zero_v[i, pl.ds(j * 16, 16)] = zero16
            return 0
        lax.fori_loop(0, _CH, fill, 0)

        _row_loop(s, lambda r0: pltpu.sync_copy(
            zero_v.at[pl.ds(0, _RCH), :], deg_sh.at[pl.ds(r0, _RCH), :]))
        plsc.subcore_barrier()

        base, count = _tile_chunks(wid)
        pltpu.sync_copy(dst_hbm.at[pl.ds(base, _TROWS), :], idx_d)

        def group(p, _):
            hs = [pltpu.async_copy(ones_v, deg_sh.at[idx_d.at[p * _DGRP + b]],
                                   ssem.at[b], add=True)
                  for b in range(_DGRP)]
            for h in hs:
                h.wait()
            return 0
        lax.fori_loop(0, count // _DGRP, group, 0)
        plsc.subcore_barrier()

        _row_loop(s, lambda r0: pltpu.sync_copy(
            deg_sh.at[pl.ds(r0, _RCH), :], deg_out.at[c, pl.ds(r0, _RCH), :]))

    return pl.kernel(body, mesh=mesh, out_type=out_type,
                     scratch_types=scratch)(dst2)[0]


def _edge_agg(m, src2, dst2):
    mesh = plsc.VectorSubcoreMesh(core_axis_name="c", subcore_axis_name="s")
    out_type = [jax.ShapeDtypeStruct((_NC, _N, _H), _f32)]
    hrows = _TROWS // 2
    scratch = [
        pltpu.VMEM((2, _CH, _H), _f32),
        pltpu.VMEM((hrows, _CH), jnp.int32),
        pltpu.VMEM((hrows, _CH), jnp.int32),
        pltpu.VMEM_SHARED((_N, _H), _f32),
        pltpu.SemaphoreType.DMA((2,)),
        pltpu.SemaphoreType.DMA((2,)),
    ]

    def body(m_hbm, src_hbm, dst_hbm, agg_out, rows_v, idx_s, idx_d, agg_sh,
             gsem, ssem):
        c = lax.axis_index("c")
        s = lax.axis_index("s")
        wid = _worker_id()
        zero16 = jnp.zeros((16,), _f32)

        def fill(i, _):
            for j in range(_H // 16):
                rows_v[0, i, pl.ds(j * 16, 16)] = zero16
            return 0
        lax.fori_loop(0, _CH, fill, 0)

        _row_loop(s, lambda r0: pltpu.sync_copy(
            rows_v.at[0, pl.ds(0, _RCH), :], agg_sh.at[pl.ds(r0, _RCH), :]))
        plsc.subcore_barrier()

        base, count = _tile_chunks(wid)

        def drain_scatter(b):
            pltpu.make_async_copy(m_hbm.at[pl.ds(0, _CH), :], rows_v.at[b],
                                  ssem.at[b]).wait()

        for hh in range(2):
            bh = pl.multiple_of(base + hh * hrows, 8)
            cnt = jnp.minimum(jnp.maximum(count - hh * hrows, 0), hrows)
            pltpu.sync_copy(src_hbm.at[pl.ds(bh, hrows), :], idx_s)
            pltpu.sync_copy(dst_hbm.at[pl.ds(bh, hrows), :], idx_d)

            def group(p, _):
                gs = []
                for b in range(2):
                    @pl.when(p > 0)
                    def _():
                        drain_scatter(b)
                    gs.append(pltpu.async_copy(m_hbm.at[idx_s.at[p * 2 + b]],
                                               rows_v.at[b], gsem.at[b]))
                for b in range(2):
                    gs[b].wait()
                    pltpu.async_copy(rows_v.at[b],
                                     agg_sh.at[idx_d.at[p * 2 + b]],
                                     ssem.at[b], add=True)
                return 0
            lax.fori_loop(0, cnt // 2, group, 0)
            @pl.when(cnt > 0)
            def _():
                for b in range(2):
                    drain_scatter(b)
        plsc.subcore_barrier()

        _row_loop(s, lambda r0: pltpu.sync_copy(
            agg_sh.at[pl.ds(r0, _RCH), :], agg_out.at[c, pl.ds(r0, _RCH), :]))

    return pl.kernel(body, mesh=mesh, out_type=out_type,
                     scratch_types=scratch)(m, src2, dst2)[0]


def _pair_gather(h, pu2, pv2):
    mesh = plsc.VectorSubcoreMesh(core_axis_name="c", subcore_axis_name="s")
    out_type = [jax.ShapeDtypeStruct((_PP, _H), _f32)] * 2
    nch = _PP // _CH
    scratch = [
        pltpu.VMEM((nch, _CH), jnp.int32),
        pltpu.VMEM((nch, _CH), jnp.int32),
        pltpu.VMEM((_CH, _H), _f32),
        pltpu.VMEM((_CH, _H), _f32),
        pltpu.SemaphoreType.DMA((2,)),
    ]

    def body(h_hbm, pu_hbm, pv_hbm, u_out, v_out, idx_u, idx_v, ru_v, rv_v,
             gsem):
        wid = _worker_id()
        pltpu.sync_copy(pu_hbm, idx_u)
        pltpu.sync_copy(pv_hbm, idx_v)
        count = 2 + (wid < (nch - 2 * _NW)).astype(jnp.int32)

        def chunk(g, _):
            j = wid + _NW * g
            hu = pltpu.async_copy(h_hbm.at[idx_u.at[j]], ru_v, gsem.at[0])
            hv = pltpu.async_copy(h_hbm.at[idx_v.at[j]], rv_v, gsem.at[1])
            e0 = pl.multiple_of(j * _CH, _CH)
            hu.wait()
            pltpu.sync_copy(ru_v, u_out.at[pl.ds(e0, _CH), :])
            hv.wait()
            pltpu.sync_copy(rv_v, v_out.at[pl.ds(e0, _CH), :])
            return 0
        lax.fori_loop(0, count, chunk, 0)

    return pl.kernel(body, mesh=mesh, out_type=out_type,
                     scratch_types=scratch)(h, pu2, pv2)


def kernel(x, edge_index, edge_type, edge_pairs, W_rel0, W_root0, b0,
           W_rel1, W_root1, b1, Wl1, bl1, Wl2, bl2, Wl3, bl3):
    del edge_type
    npad = _NW * _TROWS * _CH - _E
    zpad = jnp.zeros((npad,), jnp.int32)
    src2 = jnp.concatenate([edge_index[0], zpad]).reshape(_NW * _TROWS, _CH)
    dst2 = jnp.concatenate([edge_index[1], zpad]).reshape(_NW * _TROWS, _CH)
    pp = jnp.zeros((_PP, 2), jnp.int32).at[:_P].set(edge_pairs)
    pu2 = pp[:, 0].reshape(_PP // _CH, _CH)
    pv2 = pp[:, 1].reshape(_PP // _CH, _CH)

    deg = _deg_count(dst2)
    m0, r0 = _pre(x, W_rel0[0], W_root0, b0.reshape(1, _H))
    agg0 = _edge_agg(m0, src2, dst2)
    m1, r1 = _mid(r0, agg0, deg, W_rel1[0], W_root1, b1.reshape(1, _H))
    agg1 = _edge_agg(m1, src2, dst2)
    h2 = _fin(r1, agg1, deg)
    u, v = _pair_gather(h2, pu2, pv2)
    z = _mlp(u, v, Wl1[:_H], Wl1[_H:], bl1.reshape(1, _H),
             Wl2, bl2.reshape(1, 64), Wl3.reshape(1, 64), bl3.reshape(1, 1))
    return z[:_P, 0]


# speedup vs baseline: 7.0687x; 1.0108x over previous
import jax
import jax.numpy as jnp
from jax import lax
from jax.experimental import pallas as pl
from jax.experimental.pallas import tpu as pltpu
from jax.experimental.pallas import tpu_sc as plsc

_N = 10000
_E = 320000
_D = 128
_H = 128
_P = 10000
_PP = 10240
_CH = 128
_NC = 2
_NS = 16
_NW = _NC * _NS
_RCH = 16
_NRC = _N // _RCH
_TROWS = 80
_GRP = 4
_DGRP = 10
_BLK = 1000
_PBLK = 1024

_f32 = jnp.float32


def _pre_body(x_ref, wa_ref, wb_ref, b_ref, m_ref, r_ref):
    xb = x_ref[...]
    m_ref[...] = jnp.dot(xb, wa_ref[...], preferred_element_type=_f32)
    r_ref[...] = jnp.dot(xb, wb_ref[...], preferred_element_type=_f32) + b_ref[...]


def _pre(x, wrel, wroot, b):
    return pl.pallas_call(
        _pre_body,
        grid=(_N // _BLK,),
        in_specs=[
            pl.BlockSpec((_BLK, _D), lambda i: (i, 0)),
            pl.BlockSpec((_D, _H), lambda i: (0, 0)),
            pl.BlockSpec((_D, _H), lambda i: (0, 0)),
            pl.BlockSpec((1, _H), lambda i: (0, 0)),
        ],
        out_specs=[pl.BlockSpec((_BLK, _H), lambda i: (i, 0))] * 2,
        out_shape=[jax.ShapeDtypeStruct((_N, _H), _f32)] * 2,
    )(x, wrel, wroot, b)


def _norm_h(r_ref, agg_ref, deg_ref):
    agg = agg_ref[0] + agg_ref[1]
    deg = deg_ref[0, :, 0:1] + deg_ref[1, :, 0:1]
    inv = 1.0 / jnp.maximum(deg, 1.0)
    return jnp.maximum(r_ref[...] + agg * inv, 0.0)


def _mid_body(r_ref, agg_ref, deg_ref, wa_ref, wb_ref, b_ref, m_ref, rn_ref):
    h = _norm_h(r_ref, agg_ref, deg_ref)
    m_ref[...] = jnp.dot(h, wa_ref[...], preferred_element_type=_f32)
    rn_ref[...] = jnp.dot(h, wb_ref[...], preferred_element_type=_f32) + b_ref[...]


def _mid(r0, agg, deg, wrel, wroot, b):
    return pl.pallas_call(
        _mid_body,
        grid=(_N // _BLK,),
        in_specs=[
            pl.BlockSpec((_BLK, _H), lambda i: (i, 0)),
            pl.BlockSpec((_NC, _BLK, _H), lambda i: (0, i, 0)),
            pl.BlockSpec((_NC, _BLK, _H), lambda i: (0, i, 0)),
            pl.BlockSpec((_H, _H), lambda i: (0, 0)),
            pl.BlockSpec((_H, _H), lambda i: (0, 0)),
            pl.BlockSpec((1, _H), lambda i: (0, 0)),
        ],
        out_specs=[pl.BlockSpec((_BLK, _H), lambda i: (i, 0))] * 2,
        out_shape=[jax.ShapeDtypeStruct((_N, _H), _f32)] * 2,
    )(r0, agg, deg, wrel, wroot, b)


def _fin_body(r_ref, agg_ref, deg_ref, h_ref):
    h_ref[...] = _norm_h(r_ref, agg_ref, deg_ref)


def _fin(r1, agg, deg):
    return pl.pallas_call(
        _fin_body,
        grid=(_N // _BLK,),
        in_specs=[
            pl.BlockSpec((_BLK, _H), lambda i: (i, 0)),
            pl.BlockSpec((_NC, _BLK, _H), lambda i: (0, i, 0)),
            pl.BlockSpec((_NC, _BLK, _H), lambda i: (0, i, 0)),
        ],
        out_specs=pl.BlockSpec((_BLK, _H), lambda i: (i, 0)),
        out_shape=jax.ShapeDtypeStruct((_N, _H), _f32),
    )(r1, agg, deg)


def _mlp_body(u_ref, v_ref, w1u_ref, w1v_ref, b1_ref, w2_ref, b2_ref,
              w3_ref, b3_ref, o_ref):
    z = jnp.maximum(
        jnp.dot(u_ref[...], w1u_ref[...], preferred_element_type=_f32)
        + jnp.dot(v_ref[...], w1v_ref[...], preferred_element_type=_f32)
        + b1_ref[...], 0.0)
    z = jnp.maximum(jnp.dot(z, w2_ref[...], preferred_element_type=_f32)
                    + b2_ref[...], 0.0)
    t = jnp.sum(z * w3_ref[...], axis=1, keepdims=True) + b3_ref[...]
    o_ref[...] = 1.0 / (1.0 + jnp.exp(-t))


def _mlp(u, v, w1u, w1v, b1, w2, b2, w3r, b3):
    return pl.pallas_call(
        _mlp_body,
        grid=(_PP // _PBLK,),
        in_specs=[
            pl.BlockSpec((_PBLK, _H), lambda i: (i, 0)),
            pl.BlockSpec((_PBLK, _H), lambda i: (i, 0)),
            pl.BlockSpec((_H, _H), lambda i: (0, 0)),
            pl.BlockSpec((_H, _H), lambda i: (0, 0)),
            pl.BlockSpec((1, _H), lambda i: (0, 0)),
            pl.BlockSpec((_H, 64), lambda i: (0, 0)),
            pl.BlockSpec((1, 64), lambda i: (0, 0)),
            pl.BlockSpec((1, 64), lambda i: (0, 0)),
            pl.BlockSpec((1, 1), lambda i: (0, 0)),
        ],
        out_specs=pl.BlockSpec((_PBLK, 1), lambda i: (i, 0)),
        out_shape=jax.ShapeDtypeStruct((_PP, 1), _f32),
    )(u, v, w1u, w1v, b1, w2, b2, w3r, b3)


def _worker_id():
    return lax.axis_index("s") * _NC + lax.axis_index("c")


def _split(nch, nworkers, wid):
    nbase, extra = nch // nworkers, nch % nworkers
    start = nbase * wid + jnp.minimum(wid, extra)
    count = nbase + (wid < extra).astype(jnp.int32)
    return start, count


def _chunk_range(nch, wid):
    return _split(nch, _NW, wid)


def _row_loop(s, fn):
    start, count = _split(_NRC, _NS, s)

    def body(k, _):
        fn(pl.multiple_of((start + k) * _RCH, _RCH))
        return 0
    lax.fori_loop(0, count, body, 0)


def _tile_chunks(wid):
    base = pl.multiple_of(wid * _TROWS, _TROWS)
    count = jnp.minimum(jnp.maximum(_E // _CH - wid * _TROWS, 0), _TROWS)
    return base, count


def _deg_agg(m, src2, dst2):
    mesh = plsc.VectorSubcoreMesh(core_axis_name="c", subcore_axis_name="s")
    out_type = [
        jax.ShapeDtypeStruct((_NC, _N, _H), _f32),
        jax.ShapeDtypeStruct((_NC, _N, _H), _f32),
    ]
    hrows = _TROWS // 2
    scratch = [
        pltpu.VMEM((2, _CH, _H), _f32),
        pltpu.VMEM((hrows, _CH), jnp.int32),
        pltpu.VMEM((_TROWS, _CH), jnp.int32),
        pltpu.VMEM_SHARED((_N, _H), _f32),
        pltpu.SemaphoreType.DMA((2,)),
        pltpu.SemaphoreType.DMA((_DGRP,)),
    ]

    def body(m_hbm, src_hbm, dst_hbm, agg_out, deg_out, rows_v, idx_s, idx_d,
             agg_sh, gsem, ssem):
        c = lax.axis_index("c")
        s = lax.axis_index("s")
        wid = _worker_id()
        zero16 = jnp.zeros((16,), _f32)
        one16 = jnp.ones((16,), _f32)

        def fill(i, _):
            for j in range(_H // 16):
                rows_v[0, i, pl.ds(j * 16, 16)] = one16
                rows_v[1, i, pl.ds(j * 16, 16)] = zero16
            return 0
        lax.fori_loop(0, _CH, fill, 0)

        def zero_acc():
            _row_loop(s, lambda r0: pltpu.sync_copy(
                rows_v.at[1, pl.ds(0, _RCH), :], agg_sh.at[pl.ds(r0, _RCH), :]))

        zero_acc()
        base, count = _tile_chunks(wid)
        pltpu.sync_copy(dst_hbm.at[pl.ds(base, _TROWS), :], idx_d)
        plsc.subcore_barrier()

        def dgroup(p, _):
            hs = [pltpu.async_copy(rows_v.at[0],
                                   agg_sh.at[idx_d.at[p * _DGRP + b]],
                                   ssem.at[b], add=True)
                  for b in range(_DGRP)]
            for h in hs:
                h.wait()
            return 0
        lax.fori_loop(0, count // _DGRP, dgroup, 0)
        plsc.subcore_barrier()
        _row_loop(s, lambda r0: pltpu.sync_copy(
            agg_sh.at[pl.ds(r0, _RCH), :], deg_out.at[c, pl.ds(r0, _RCH), :]))
        plsc.subcore_barrier()
        zero_acc()
        plsc.subcore_barrier()

        def drain_scatter(b):
            pltpu.make_async_copy(m_hbm.at[pl.ds(0, _CH), :], rows_v.at[b],
                                  ssem.at[b]).wait()

        for hh in range(2):
            bh = pl.multiple_of(base + hh * hrows, 8)
            cnt = jnp.minimum(jnp.maximum(count - hh * hrows, 0), hrows)
            pltpu.sync_copy(src_hbm.at[pl.ds(bh, hrows), :], idx_s)

            def group(p, _):
                gs = []
                for b in range(2):
                    @pl.when(p > 0)
                    def _():
                        drain_scatter(b)
                    gs.append(pltpu.async_copy(m_hbm.at[idx_s.at[p * 2 + b]],
                                               rows_v.at[b], gsem.at[b]))
                for b in range(2):
                    gs[b].wait()
                    pltpu.async_copy(
                        rows_v.at[b],
                        agg_sh.at[idx_d.at[hh * hrows + p * 2 + b]],
                        ssem.at[b], add=True)
                return 0
            lax.fori_loop(0, cnt // 2, group, 0)

            @pl.when(cnt > 0)
            def _():
                for b in range(2):
                    drain_scatter(b)
        plsc.subcore_barrier()
        _row_loop(s, lambda r0: pltpu.sync_copy(
            agg_sh.at[pl.ds(r0, _RCH), :], agg_out.at[c, pl.ds(r0, _RCH), :]))

    return pl.kernel(body, mesh=mesh, out_type=out_type,
                     scratch_types=scratch)(m, src2, dst2)


def _edge_agg(m, src2, dst2):
    mesh = plsc.VectorSubcoreMesh(core_axis_name="c", subcore_axis_name="s")
    out_type = [jax.ShapeDtypeStruct((_NC, _N, _H), _f32)]
    hrows = _TROWS // 2
    scratch = [
        pltpu.VMEM((2, _CH, _H), _f32),
        pltpu.VMEM((hrows, _CH), jnp.int32),
        pltpu.VMEM((hrows, _CH), jnp.int32),
        pltpu.VMEM_SHARED((_N, _H), _f32),
        pltpu.SemaphoreType.DMA((2,)),
        pltpu.SemaphoreType.DMA((2,)),
    ]

    def body(m_hbm, src_hbm, dst_hbm, agg_out, rows_v, idx_s, idx_d, agg_sh,
             gsem, ssem):
        c = lax.axis_index("c")
        s = lax.axis_index("s")
        wid = _worker_id()
        zero16 = jnp.zeros((16,), _f32)

        def fill(i, _):
            for j in range(_H // 16):
                rows_v[0, i, pl.ds(j * 16, 16)] = zero16
            return 0
        lax.fori_loop(0, _CH, fill, 0)

        _row_loop(s, lambda r0: pltpu.sync_copy(
            rows_v.at[0, pl.ds(0, _RCH), :], agg_sh.at[pl.ds(r0, _RCH), :]))
        plsc.subcore_barrier()

        base, count = _tile_chunks(wid)

        def drain_scatter(b):
            pltpu.make_async_copy(m_hbm.at[pl.ds(0, _CH), :], rows_v.at[b],
                                  ssem.at[b]).wait()

        for hh in range(2):
            bh = pl.multiple_of(base + hh * hrows, 8)
            cnt = jnp.minimum(jnp.maximum(count - hh * hrows, 0), hrows)
            pltpu.sync_copy(src_hbm.at[pl.ds(bh, hrows), :], idx_s)
            pltpu.sync_copy(dst_hbm.at[pl.ds(bh, hrows), :], idx_d)

            def group(p, _):
                gs = []
                for b in range(2):
                    @pl.when(p > 0)
                    def _():
                        drain_scatter(b)
                    gs.append(pltpu.async_copy(m_hbm.at[idx_s.at[p * 2 + b]],
                                               rows_v.at[b], gsem.at[b]))
                for b in range(2):
                    gs[b].wait()
                    pltpu.async_copy(rows_v.at[b],
                                     agg_sh.at[idx_d.at[p * 2 + b]],
                                     ssem.at[b], add=True)
                return 0
            lax.fori_loop(0, cnt // 2, group, 0)
            @pl.when(cnt > 0)
            def _():
                for b in range(2):
                    drain_scatter(b)
        plsc.subcore_barrier()

        _row_loop(s, lambda r0: pltpu.sync_copy(
            agg_sh.at[pl.ds(r0, _RCH), :], agg_out.at[c, pl.ds(r0, _RCH), :]))

    return pl.kernel(body, mesh=mesh, out_type=out_type,
                     scratch_types=scratch)(m, src2, dst2)[0]


def _pair_gather(h, pu2, pv2):
    mesh = plsc.VectorSubcoreMesh(core_axis_name="c", subcore_axis_name="s")
    out_type = [jax.ShapeDtypeStruct((_PP, _H), _f32)] * 2
    nch = _PP // _CH
    scratch = [
        pltpu.VMEM((nch, _CH), jnp.int32),
        pltpu.VMEM((nch, _CH), jnp.int32),
        pltpu.VMEM((_CH, _H), _f32),
        pltpu.VMEM((_CH, _H), _f32),
        pltpu.SemaphoreType.DMA((2,)),
    ]

    def body(h_hbm, pu_hbm, pv_hbm, u_out, v_out, idx_u, idx_v, ru_v, rv_v,
             gsem):
        wid = _worker_id()
        pltpu.sync_copy(pu_hbm, idx_u)
        pltpu.sync_copy(pv_hbm, idx_v)
        count = 2 + (wid < (nch - 2 * _NW)).astype(jnp.int32)

        def chunk(g, _):
            j = wid + _NW * g
            hu = pltpu.async_copy(h_hbm.at[idx_u.at[j]], ru_v, gsem.at[0])
            hv = pltpu.async_copy(h_hbm.at[idx_v.at[j]], rv_v, gsem.at[1])
            e0 = pl.multiple_of(j * _CH, _CH)
            hu.wait()
            pltpu.sync_copy(ru_v, u_out.at[pl.ds(e0, _CH), :])
            hv.wait()
            pltpu.sync_copy(rv_v, v_out.at[pl.ds(e0, _CH), :])
            return 0
        lax.fori_loop(0, count, chunk, 0)

    return pl.kernel(body, mesh=mesh, out_type=out_type,
                     scratch_types=scratch)(h, pu2, pv2)


def kernel(x, edge_index, edge_type, edge_pairs, W_rel0, W_root0, b0,
           W_rel1, W_root1, b1, Wl1, bl1, Wl2, bl2, Wl3, bl3):
    del edge_type
    npad = _NW * _TROWS * _CH - _E
    zpad = jnp.zeros((npad,), jnp.int32)
    src2 = jnp.concatenate([edge_index[0], zpad]).reshape(_NW * _TROWS, _CH)
    dst2 = jnp.concatenate([edge_index[1], zpad]).reshape(_NW * _TROWS, _CH)
    pp = jnp.zeros((_PP, 2), jnp.int32).at[:_P].set(edge_pairs)
    pu2 = pp[:, 0].reshape(_PP // _CH, _CH)
    pv2 = pp[:, 1].reshape(_PP // _CH, _CH)

    m0, r0 = _pre(x, W_rel0[0], W_root0, b0.reshape(1, _H))
    agg0, deg = _deg_agg(m0, src2, dst2)
    m1, r1 = _mid(r0, agg0, deg, W_rel1[0], W_root1, b1.reshape(1, _H))
    agg1 = _edge_agg(m1, src2, dst2)
    h2 = _fin(r1, agg1, deg)
    u, v = _pair_gather(h2, pu2, pv2)
    z = _mlp(u, v, Wl1[:_H], Wl1[_H:], bl1.reshape(1, _H),
             Wl2, bl2.reshape(1, 64), Wl3.reshape(1, 64), bl3.reshape(1, 1))
    return z[:_P, 0]
